# baseline (speedup 1.0000x reference)
import jax
import jax.numpy as jnp
from jax import lax
from jax.experimental import pallas as pl
from jax.experimental.pallas import tpu as pltpu
from jax.experimental.pallas import tpu_sc as plsc

NC = 2
NS = 16
NW = NC * NS
BATCH = 16384
D = 32
BPW = BATCH // NW


def _body(uidx, iidx, uid_table, iid_table, out,
          idx_u, idx_i, rows_u, rows_i, sem_u, sem_i):
    wid = lax.axis_index("s") * NC + lax.axis_index("c")
    base = wid * BPW
    pltpu.sync_copy(uidx.at[pl.ds(base, BPW)], idx_u)
    pltpu.sync_copy(iidx.at[pl.ds(base, BPW)], idx_i)
    cu = pltpu.async_copy(uid_table.at[idx_u], rows_u, sem_u)
    ci = pltpu.async_copy(iid_table.at[idx_i], rows_i, sem_i)
    cu.wait()
    pltpu.sync_copy(rows_u, out.at[pl.ds(base, BPW), 0])
    ci.wait()
    pltpu.sync_copy(rows_i, out.at[pl.ds(base, BPW), 1])


def kernel(x, uid_table, iid_table):
    uidx = x[:, 0]
    iidx = x[:, 1]
    f = pl.kernel(
        _body,
        out_type=jax.ShapeDtypeStruct((BATCH, 2, D), jnp.float32),
        mesh=plsc.VectorSubcoreMesh(core_axis_name="c", subcore_axis_name="s"),
        scratch_types=[
            pltpu.VMEM((BPW,), jnp.int32),
            pltpu.VMEM((BPW,), jnp.int32),
            pltpu.VMEM((BPW, D), jnp.float32),
            pltpu.VMEM((BPW, D), jnp.float32),
            pltpu.SemaphoreType.DMA,
            pltpu.SemaphoreType.DMA,
        ],
        compiler_params=pltpu.CompilerParams(use_tc_tiling_on_sc=False),
    )
    return f(uidx, iidx, uid_table, iid_table)


# speedup vs baseline: 3.4903x; 3.4903x over previous
import jax
import jax.numpy as jnp
from jax import lax
from jax.experimental import pallas as pl
from jax.experimental.pallas import tpu as pltpu
from jax.experimental.pallas import tpu_sc as plsc

NC = 2
NS = 16
NW = NC * NS
BATCH = 16384
D = 32
DB = D // 8
BPW = BATCH // NW
CH = 16


def _gather_chunk(idx_ref, tab, bufs, rows, sem, g):
    v = idx_ref[pl.ds(g * CH, CH)]
    cps = []
    for j in range(CH):
        r = v[j]
        ro = pl.multiple_of((r >> 7) << 7, 128)
        for db in range(DB):
            cps.append(pltpu.async_copy(
                tab.at[db, :, pl.ds(ro, 128)], bufs.at[j, db], sem))
    for cp in cps:
        cp.wait()
    lane = lax.iota(jnp.int32, 16)
    sub = lane >> 3
    row8 = lane & 7
    for j in range(CH):
        m = v[j] & 127
        mv = jnp.full((16,), 0, jnp.int32) + m
        i_abs = jnp.full((16,), 0, jnp.int32) + (g * CH + j)
        lo = plsc.load_gather(bufs.at[j], [sub, row8, mv])
        hi = plsc.load_gather(bufs.at[j], [sub + 2, row8, mv])
        plsc.store_scatter(rows, [lane, i_abs], lo)
        plsc.store_scatter(rows, [lane + 16, i_abs], hi)


def _body(uidx, iidx, tab_u, tab_i, out,
          idx_u, idx_i, bufs, rows_u, rows_i, sem):
    wid = lax.axis_index("s") * NC + lax.axis_index("c")
    base = wid * BPW
    pltpu.sync_copy(uidx.at[pl.ds(base, BPW)], idx_u)
    pltpu.sync_copy(iidx.at[pl.ds(base, BPW)], idx_i)

    def chunk_u(g, carry):
        _gather_chunk(idx_u, tab_u, bufs, rows_u, sem, g)
        return carry

    def chunk_i(g, carry):
        _gather_chunk(idx_i, tab_i, bufs, rows_i, sem, g)
        return carry

    lax.fori_loop(0, BPW // CH, chunk_u, 0)
    lax.fori_loop(0, BPW // CH, chunk_i, 0)
    pltpu.sync_copy(rows_u, out.at[0, :, pl.ds(base, BPW)])
    pltpu.sync_copy(rows_i, out.at[1, :, pl.ds(base, BPW)])


def kernel(x, uid_table, iid_table):
    uidx = x[:, 0]
    iidx = x[:, 1]
    tab_u = uid_table.T.reshape(DB, 8, uid_table.shape[0])
    tab_i = iid_table.T.reshape(DB, 8, iid_table.shape[0])
    f = pl.kernel(
        _body,
        out_type=jax.ShapeDtypeStruct((2, D, BATCH), jnp.float32),
        mesh=plsc.VectorSubcoreMesh(core_axis_name="c", subcore_axis_name="s"),
        scratch_types=[
            pltpu.VMEM((BPW,), jnp.int32),
            pltpu.VMEM((BPW,), jnp.int32),
            pltpu.VMEM((CH, DB, 8, 128), jnp.float32),
            pltpu.VMEM((D, BPW), jnp.float32),
            pltpu.VMEM((D, BPW), jnp.float32),
            pltpu.SemaphoreType.DMA,
        ],
        compiler_params=pltpu.CompilerParams(needs_layout_passes=False),
    )
    out = f(uidx, iidx, tab_u, tab_i)
    return jnp.transpose(out, (2, 0, 1))


# speedup vs baseline: 3.4965x; 1.0018x over previous
import jax
import jax.numpy as jnp
from jax import lax
from jax.experimental import pallas as pl
from jax.experimental.pallas import tpu as pltpu
from jax.experimental.pallas import tpu_sc as plsc

NC = 2
NS = 16
NW = NC * NS
BATCH = 16384
D = 32
DB = D // 8
BPW = BATCH // NW
CH = 16


def _gather_chunk(idx_ref, tab, bufs, rows, sems, g):
    v = idx_ref[pl.ds(g * CH, CH)]
    cps = []
    for j in range(CH):
        r = v[j]
        ro = pl.multiple_of((r >> 7) << 7, 128)
        cps.append(pltpu.async_copy(
            tab.at[:, :, pl.ds(ro, 128)], bufs.at[j], sems[j]))
    lane = lax.iota(jnp.int32, 16)
    sub = lane >> 3
    row8 = lane & 7
    for j in range(CH):
        cps[j].wait()
        m = v[j] & 127
        mv = jnp.full((16,), 0, jnp.int32) + m
        i_abs = jnp.full((16,), 0, jnp.int32) + (g * CH + j)
        lo = plsc.load_gather(bufs.at[j], [sub, row8, mv])
        hi = plsc.load_gather(bufs.at[j], [sub + 2, row8, mv])
        plsc.store_scatter(rows, [lane, i_abs], lo)
        plsc.store_scatter(rows, [lane + 16, i_abs], hi)


def _body(uidx, iidx, tab_u, tab_i, out,
          idx_u, idx_i, bufs, rows_u, rows_i, *sems):
    wid = lax.axis_index("s") * NC + lax.axis_index("c")
    base = wid * BPW
    pltpu.sync_copy(uidx.at[pl.ds(base, BPW)], idx_u)
    pltpu.sync_copy(iidx.at[pl.ds(base, BPW)], idx_i)

    def chunk_u(g, carry):
        _gather_chunk(idx_u, tab_u, bufs, rows_u, sems, g)
        return carry

    def chunk_i(g, carry):
        _gather_chunk(idx_i, tab_i, bufs, rows_i, sems, g)
        return carry

    lax.fori_loop(0, BPW // CH, chunk_u, 0)
    lax.fori_loop(0, BPW // CH, chunk_i, 0)
    pltpu.sync_copy(rows_u, out.at[0, :, pl.ds(base, BPW)])
    pltpu.sync_copy(rows_i, out.at[1, :, pl.ds(base, BPW)])


def kernel(x, uid_table, iid_table):
    uidx = x[:, 0]
    iidx = x[:, 1]
    tab_u = uid_table.T.reshape(DB, 8, uid_table.shape[0])
    tab_i = iid_table.T.reshape(DB, 8, iid_table.shape[0])
    f = pl.kernel(
        _body,
        out_type=jax.ShapeDtypeStruct((2, D, BATCH), jnp.float32),
        mesh=plsc.VectorSubcoreMesh(core_axis_name="c", subcore_axis_name="s"),
        scratch_types=[
            pltpu.VMEM((BPW,), jnp.int32),
            pltpu.VMEM((BPW,), jnp.int32),
            pltpu.VMEM((CH, DB, 8, 128), jnp.float32),
            pltpu.VMEM((D, BPW), jnp.float32),
            pltpu.VMEM((D, BPW), jnp.float32),
        ] + [pltpu.SemaphoreType.DMA] * CH,
        compiler_params=pltpu.CompilerParams(needs_layout_passes=False),
    )
    out = f(uidx, iidx, tab_u, tab_i)
    return jnp.transpose(out, (2, 0, 1))
